# baseline (speedup 1.0000x reference)
import functools

import jax
import jax.numpy as jnp
from jax import lax
from jax.experimental import pallas as pl
from jax.experimental.pallas import tpu as pltpu
from jax.experimental.pallas import tpu_sc as plsc

MAX_LEN = 2
NUM_HEADS = 8
FEAT_DIM = 128
N_GRAPH = 16
MAX_NODES = 128
EDGES_PER_GRAPH = 2048

_NC = 2
_NS = 16
_NW = _NC * _NS
_PAIRS = MAX_NODES * MAX_NODES
_HALF = _PAIRS // 2
_PROJ_W = MAX_LEN * NUM_HEADS
_GSLICE = EDGES_PER_GRAPH * _PROJ_W
_OUT_W = _HALF * NUM_HEADS


def _proj_matmul(edge_feat, emb_weight):
    blk = 4096

    def body(x_ref, w_ref, o_ref):
        o_ref[...] = lax.dot_general(
            x_ref[...], w_ref[...],
            (((1,), (1,)), ((), ())),
            preferred_element_type=jnp.float32,
        )

    return pl.pallas_call(
        body,
        grid=(N_GRAPH * EDGES_PER_GRAPH // blk,),
        in_specs=[
            pl.BlockSpec((blk, FEAT_DIM), lambda i: (i, 0)),
            pl.BlockSpec((_PROJ_W, FEAT_DIM), lambda i: (0, 0)),
        ],
        out_specs=pl.BlockSpec((blk, _PROJ_W), lambda i: (i, 0)),
        out_shape=jax.ShapeDtypeStruct((N_GRAPH * EDGES_PER_GRAPH, _PROJ_W),
                                       jnp.float32),
    )(edge_feat, emb_weight)


_OUT_ROWS = N_GRAPH * _PAIRS * NUM_HEADS // FEAT_DIM
_CHUNK_ROWS = _OUT_W // FEAT_DIM


def _sc_combine(proj_flat, p0, p1, dist_flat):
    mesh = plsc.VectorSubcoreMesh(
        core_axis_name="c", subcore_axis_name="s",
        num_cores=_NC, num_subcores=_NS)

    @functools.partial(
        pl.kernel,
        out_type=jax.ShapeDtypeStruct((_OUT_ROWS, FEAT_DIM), jnp.float32),
        mesh=mesh,
        compiler_params=pltpu.CompilerParams(needs_layout_passes=False),
        scratch_types=[
            pltpu.VMEM((_GSLICE,), jnp.float32),
            pltpu.VMEM((_HALF,), jnp.int32),
            pltpu.VMEM((_HALF,), jnp.int32),
            pltpu.VMEM((_HALF,), jnp.int32),
            pltpu.VMEM((_CHUNK_ROWS, FEAT_DIM), jnp.float32),
        ],
    )
    def k(proj_hbm, p0_hbm, p1_hbm, dist_hbm, out_hbm,
          projv, p0v, p1v, dv, outv):
        wid = lax.axis_index("s") * _NC + lax.axis_index("c")
        iota = lax.iota(jnp.int32, 16)
        iota8 = iota * NUM_HEADS

        @pl.when(wid < (N_GRAPH - 1) * 2)
        def _compute():
            i = wid // 2
            half = wid % 2
            g = i + 1
            poff = pl.multiple_of(g * _PAIRS + half * _HALF, 8)
            pltpu.sync_copy(
                proj_hbm.at[pl.ds(pl.multiple_of(i * _GSLICE, 8), _GSLICE)],
                projv)
            pltpu.sync_copy(p0_hbm.at[pl.ds(poff, _HALF)], p0v)
            pltpu.sync_copy(p1_hbm.at[pl.ds(poff, _HALF)], p1v)
            pltpu.sync_copy(dist_hbm.at[pl.ds(poff, _HALF)], dv)

            def body(b, _):
                base = b * 16
                s0 = p0v[pl.ds(base, 16)] * _PROJ_W
                s1 = p1v[pl.ds(base, 16)] * _PROJ_W + NUM_HEADS
                dvec = dv[pl.ds(base, 16)]
                rvec = jnp.where(dvec >= 2, jnp.float32(0.5), jnp.float32(1.0))
                row = jnp.broadcast_to(b, (16,))
                for h in range(NUM_HEADS):
                    av = plsc.load_gather(projv, [s0 + h])
                    bv = plsc.load_gather(projv, [s1 + h])
                    plsc.store_scatter(outv, [row, iota8 + h],
                                       (av + bv) * rvec)
                return _

            lax.fori_loop(0, _CHUNK_ROWS, body, None)
            pltpu.sync_copy(
                outv,
                out_hbm.at[pl.ds(
                    pl.multiple_of(i * 2 * _CHUNK_ROWS + half * _CHUNK_ROWS,
                                   8),
                    _CHUNK_ROWS)])

        @pl.when(wid >= _NW - 2)
        def _fill():
            half = wid - (_NW - 2)
            neg = jnp.full((16,), -1000.0, dtype=jnp.float32)

            def body(b, _):
                for c in range(FEAT_DIM // 16):
                    outv[b, pl.ds(c * 16, 16)] = neg
                return _

            lax.fori_loop(0, _CHUNK_ROWS, body, None)
            pltpu.sync_copy(
                outv,
                out_hbm.at[pl.ds(
                    pl.multiple_of((N_GRAPH - 1) * 2 * _CHUNK_ROWS
                                   + half * _CHUNK_ROWS, 8),
                    _CHUNK_ROWS)])

    return k(proj_flat, p0, p1, dist_flat)


def kernel(edge_feat, dist, path, emb_weight):
    proj = _proj_matmul(edge_feat, emb_weight)
    proj_flat = proj.reshape(-1)
    p0 = path[:, :, :, 0].reshape(-1)
    p1 = path[:, :, :, 1].reshape(-1)
    dist_flat = dist.reshape(-1)
    out = _sc_combine(proj_flat, p0, p1, dist_flat)
    return out.reshape(N_GRAPH, MAX_NODES, MAX_NODES, NUM_HEADS)


# speedup vs baseline: 2.0299x; 2.0299x over previous
import functools

import jax
import jax.numpy as jnp
from jax import lax
from jax.experimental import pallas as pl
from jax.experimental.pallas import tpu as pltpu
from jax.experimental.pallas import tpu_sc as plsc

MAX_LEN = 2
NUM_HEADS = 8
FEAT_DIM = 128
N_GRAPH = 16
MAX_NODES = 128
EDGES_PER_GRAPH = 2048

_NC = 2
_NS = 16
_NW = _NC * _NS
_PROJ_W = MAX_LEN * NUM_HEADS
_GSLICE = EDGES_PER_GRAPH * _PROJ_W
_XH = MAX_NODES // 2
_CHUNK_ROWS = _XH * NUM_HEADS
_OUT_ROWS = N_GRAPH * MAX_NODES * NUM_HEADS


def _proj_matmul(edge_feat, emb_weight):
    blk = 4096

    def body(x_ref, w_ref, o_ref):
        o_ref[...] = lax.dot_general(
            x_ref[...], w_ref[...],
            (((1,), (1,)), ((), ())),
            preferred_element_type=jnp.float32,
        )

    return pl.pallas_call(
        body,
        grid=(N_GRAPH * EDGES_PER_GRAPH // blk,),
        in_specs=[
            pl.BlockSpec((blk, FEAT_DIM), lambda i: (i, 0)),
            pl.BlockSpec((_PROJ_W, FEAT_DIM), lambda i: (0, 0)),
        ],
        out_specs=pl.BlockSpec((blk, _PROJ_W), lambda i: (i, 0)),
        out_shape=jax.ShapeDtypeStruct((N_GRAPH * EDGES_PER_GRAPH, _PROJ_W),
                                       jnp.float32),
    )(edge_feat, emb_weight)


def _sc_combine(proj_flat, path_t, dist):
    mesh = plsc.VectorSubcoreMesh(
        core_axis_name="c", subcore_axis_name="s",
        num_cores=_NC, num_subcores=_NS)

    @functools.partial(
        pl.kernel,
        out_type=jax.ShapeDtypeStruct((_OUT_ROWS, FEAT_DIM), jnp.float32),
        mesh=mesh,
        compiler_params=pltpu.CompilerParams(needs_layout_passes=False),
        scratch_types=[
            pltpu.VMEM((_GSLICE,), jnp.float32),
            pltpu.VMEM((_XH, 2, FEAT_DIM), jnp.int32),
            pltpu.VMEM((_XH, FEAT_DIM), jnp.int32),
            pltpu.VMEM((_CHUNK_ROWS, FEAT_DIM), jnp.float32),
        ],
    )
    def k(proj_hbm, path_hbm, dist_hbm, out_hbm, projv, pv, dv, outv):
        wid = lax.axis_index("s") * _NC + lax.axis_index("c")

        @pl.when(wid < (N_GRAPH - 1) * 2)
        def _compute():
            i = wid // 2
            half = wid % 2
            g = i + 1
            x0 = half * _XH
            pltpu.sync_copy(
                proj_hbm.at[pl.ds(pl.multiple_of(i * _GSLICE, 8), _GSLICE)],
                projv)
            pltpu.sync_copy(path_hbm.at[g, pl.ds(x0, _XH)], pv)
            pltpu.sync_copy(dist_hbm.at[g, pl.ds(x0, _XH)], dv)

            def body(x, _):
                for yb in range(FEAT_DIM // 16):
                    ys = yb * 16
                    s0 = pv[x, 0, pl.ds(ys, 16)] * _PROJ_W
                    s1 = pv[x, 1, pl.ds(ys, 16)] * _PROJ_W + NUM_HEADS
                    dvec = dv[x, pl.ds(ys, 16)]
                    rvec = jnp.where(dvec >= 2, jnp.float32(0.5),
                                     jnp.float32(1.0))
                    for h in range(NUM_HEADS):
                        av = plsc.load_gather(projv, [s0 + h])
                        bv = plsc.load_gather(projv, [s1 + h])
                        outv[x * NUM_HEADS + h, pl.ds(ys, 16)] = \
                            (av + bv) * rvec
                return _

            lax.fori_loop(0, _XH, body, None)
            pltpu.sync_copy(
                outv,
                out_hbm.at[pl.ds(
                    pl.multiple_of((i * MAX_NODES + x0) * NUM_HEADS, 8),
                    _CHUNK_ROWS)])

        @pl.when(wid >= _NW - 2)
        def _fill():
            half = wid - (_NW - 2)
            neg = jnp.full((16,), -1000.0, dtype=jnp.float32)

            def body(r, _):
                for c in range(FEAT_DIM // 16):
                    outv[r, pl.ds(c * 16, 16)] = neg
                return _

            lax.fori_loop(0, _CHUNK_ROWS, body, None)
            pltpu.sync_copy(
                outv,
                out_hbm.at[pl.ds(
                    pl.multiple_of(((N_GRAPH - 1) * MAX_NODES
                                    + half * _XH) * NUM_HEADS, 8),
                    _CHUNK_ROWS)])

    return k(proj_flat, path_t, dist)


def kernel(edge_feat, dist, path, emb_weight):
    proj = _proj_matmul(edge_feat, emb_weight)
    proj_flat = proj.reshape(-1)
    path_t = jnp.transpose(path, (0, 1, 3, 2))
    out2d = _sc_combine(proj_flat, path_t, dist)
    out = out2d.reshape(N_GRAPH, MAX_NODES, NUM_HEADS, MAX_NODES)
    return jnp.transpose(out, (0, 1, 3, 2))


# speedup vs baseline: 2.8340x; 1.3961x over previous
import functools

import jax
import jax.numpy as jnp
from jax import lax
from jax.experimental import pallas as pl
from jax.experimental.pallas import tpu as pltpu
from jax.experimental.pallas import tpu_sc as plsc

MAX_LEN = 2
NUM_HEADS = 8
FEAT_DIM = 128
N_GRAPH = 16
MAX_NODES = 128
EDGES_PER_GRAPH = 2048

_NC = 2
_NS = 16
_NW = _NC * _NS
_PROJ_W = MAX_LEN * NUM_HEADS
_GSLICE = EDGES_PER_GRAPH * _PROJ_W
_XH = MAX_NODES // 2
_CHUNK_ROWS = _XH * NUM_HEADS
_OUT_ROWS = N_GRAPH * MAX_NODES * NUM_HEADS


def _proj_matmul(edge_feat, emb_weight):
    blk = 4096

    def body(x_ref, w_ref, o_ref):
        o_ref[...] = lax.dot_general(
            x_ref[...], w_ref[...],
            (((1,), (1,)), ((), ())),
            preferred_element_type=jnp.float32,
        )

    return pl.pallas_call(
        body,
        grid=(N_GRAPH * EDGES_PER_GRAPH // blk,),
        in_specs=[
            pl.BlockSpec((blk, FEAT_DIM), lambda i: (i, 0)),
            pl.BlockSpec((_PROJ_W, FEAT_DIM), lambda i: (0, 0)),
        ],
        out_specs=pl.BlockSpec((blk, _PROJ_W), lambda i: (i, 0)),
        out_shape=jax.ShapeDtypeStruct((N_GRAPH * EDGES_PER_GRAPH, _PROJ_W),
                                       jnp.float32),
    )(edge_feat, emb_weight)


def _sc_combine(proj_flat, path_t, dist):
    mesh = plsc.VectorSubcoreMesh(
        core_axis_name="c", subcore_axis_name="s",
        num_cores=_NC, num_subcores=_NS)

    @functools.partial(
        pl.kernel,
        out_type=jax.ShapeDtypeStruct((_OUT_ROWS, FEAT_DIM), jnp.float32),
        mesh=mesh,
        compiler_params=pltpu.CompilerParams(needs_layout_passes=False),
        scratch_types=[
            pltpu.VMEM((_GSLICE,), jnp.float32),
            pltpu.VMEM((_XH, 2, FEAT_DIM), jnp.int32),
            pltpu.VMEM((_XH, FEAT_DIM), jnp.int32),
            pltpu.VMEM((_CHUNK_ROWS, FEAT_DIM), jnp.float32),
        ],
    )
    def k(proj_hbm, path_hbm, dist_hbm, out_hbm, projv, pv, dv, outv):
        wid = lax.axis_index("s") * _NC + lax.axis_index("c")

        @pl.when(wid < (N_GRAPH - 1) * 2)
        def _compute():
            i = wid // 2
            half = wid % 2
            g = i + 1
            x0 = half * _XH
            pltpu.sync_copy(
                proj_hbm.at[pl.ds(pl.multiple_of(i * _GSLICE, 8), _GSLICE)],
                projv)
            pltpu.sync_copy(path_hbm.at[g, pl.ds(x0, _XH)], pv)
            pltpu.sync_copy(dist_hbm.at[g, pl.ds(x0, _XH)], dv)

            @plsc.parallel_loop(0, _XH, step=1, unroll=2)
            def _body(x):
                for yb in range(FEAT_DIM // 16):
                    ys = yb * 16
                    s0 = pv[x, 0, pl.ds(ys, 16)] * _PROJ_W
                    s1 = pv[x, 1, pl.ds(ys, 16)] * _PROJ_W + NUM_HEADS
                    dvec = dv[x, pl.ds(ys, 16)]
                    rvec = jnp.where(dvec >= 2, jnp.float32(0.5),
                                     jnp.float32(1.0))
                    for h in range(NUM_HEADS):
                        av = plsc.load_gather(projv, [s0 + h])
                        bv = plsc.load_gather(projv, [s1 + h])
                        outv[x * NUM_HEADS + h, pl.ds(ys, 16)] = \
                            (av + bv) * rvec
            pltpu.sync_copy(
                outv,
                out_hbm.at[pl.ds(
                    pl.multiple_of((i * MAX_NODES + x0) * NUM_HEADS, 8),
                    _CHUNK_ROWS)])

        @pl.when(wid >= _NW - 2)
        def _fill():
            half = wid - (_NW - 2)
            neg = jnp.full((16,), -1000.0, dtype=jnp.float32)

            def body(r, _):
                for c in range(FEAT_DIM // 16):
                    outv[r, pl.ds(c * 16, 16)] = neg
                return _

            lax.fori_loop(0, _CHUNK_ROWS, body, None)
            pltpu.sync_copy(
                outv,
                out_hbm.at[pl.ds(
                    pl.multiple_of(((N_GRAPH - 1) * MAX_NODES
                                    + half * _XH) * NUM_HEADS, 8),
                    _CHUNK_ROWS)])

    return k(proj_flat, path_t, dist)


def kernel(edge_feat, dist, path, emb_weight):
    proj = _proj_matmul(edge_feat, emb_weight)
    proj_flat = proj.reshape(-1)
    path_t = jnp.transpose(path, (0, 1, 3, 2))
    out2d = _sc_combine(proj_flat, path_t, dist)
    out = out2d.reshape(N_GRAPH, MAX_NODES, NUM_HEADS, MAX_NODES)
    return jnp.transpose(out, (0, 1, 3, 2))


# speedup vs baseline: 3.2602x; 1.1504x over previous
import functools

import jax
import jax.numpy as jnp
from jax import lax
from jax.experimental import pallas as pl
from jax.experimental.pallas import tpu as pltpu
from jax.experimental.pallas import tpu_sc as plsc

MAX_LEN = 2
NUM_HEADS = 8
FEAT_DIM = 128
N_GRAPH = 16
MAX_NODES = 128
EDGES_PER_GRAPH = 2048

_NC = 2
_NS = 16
_NW = _NC * _NS
_PROJ_W = MAX_LEN * NUM_HEADS
_GSLICE = EDGES_PER_GRAPH * _PROJ_W
_XH = MAX_NODES // 2
_CHUNK_ROWS = _XH * NUM_HEADS
_OUT_ROWS = N_GRAPH * MAX_NODES * NUM_HEADS


def _proj_matmul(edge_feat, emb_weight):
    rows = N_GRAPH * EDGES_PER_GRAPH // 8
    blk = 512

    x3 = edge_feat.reshape(rows, 8, FEAT_DIM)
    bdiag = (jnp.eye(8, dtype=jnp.float32)[:, None, :, None]
             * jnp.transpose(emb_weight)[None, :, None, :]
             ).reshape(8, FEAT_DIM, 8 * _PROJ_W)

    def body(x_ref, w_ref, o_ref):
        acc = lax.dot_general(
            x_ref[:, 0, :], w_ref[0],
            (((1,), (0,)), ((), ())), preferred_element_type=jnp.float32)
        for s in range(1, 8):
            acc += lax.dot_general(
                x_ref[:, s, :], w_ref[s],
                (((1,), (0,)), ((), ())), preferred_element_type=jnp.float32)
        o_ref[...] = acc

    return pl.pallas_call(
        body,
        grid=(rows // blk,),
        in_specs=[
            pl.BlockSpec((blk, 8, FEAT_DIM), lambda i: (i, 0, 0)),
            pl.BlockSpec((8, FEAT_DIM, 8 * _PROJ_W), lambda i: (0, 0, 0)),
        ],
        out_specs=pl.BlockSpec((blk, 8 * _PROJ_W), lambda i: (i, 0)),
        out_shape=jax.ShapeDtypeStruct((rows, 8 * _PROJ_W), jnp.float32),
    )(x3, bdiag)


def _sc_combine(proj_flat, path_t, dist):
    mesh = plsc.VectorSubcoreMesh(
        core_axis_name="c", subcore_axis_name="s",
        num_cores=_NC, num_subcores=_NS)

    @functools.partial(
        pl.kernel,
        out_type=jax.ShapeDtypeStruct((_OUT_ROWS, FEAT_DIM), jnp.float32),
        mesh=mesh,
        compiler_params=pltpu.CompilerParams(needs_layout_passes=False),
        scratch_types=[
            pltpu.VMEM((_GSLICE,), jnp.float32),
            pltpu.VMEM((_XH, 2, FEAT_DIM), jnp.int32),
            pltpu.VMEM((_XH, FEAT_DIM), jnp.int32),
            pltpu.VMEM((_CHUNK_ROWS, FEAT_DIM), jnp.float32),
        ],
    )
    def k(proj_hbm, path_hbm, dist_hbm, out_hbm, projv, pv, dv, outv):
        wid = lax.axis_index("s") * _NC + lax.axis_index("c")

        @pl.when(wid < (N_GRAPH - 1) * 2)
        def _compute():
            i = wid // 2
            half = wid % 2
            g = i + 1
            x0 = half * _XH
            pltpu.sync_copy(
                proj_hbm.at[pl.ds(pl.multiple_of(i * _GSLICE, 8), _GSLICE)],
                projv)
            pltpu.sync_copy(path_hbm.at[g, pl.ds(x0, _XH)], pv)
            pltpu.sync_copy(dist_hbm.at[g, pl.ds(x0, _XH)], dv)

            @plsc.parallel_loop(0, _XH, step=1, unroll=2)
            def _body(x):
                for yb in range(FEAT_DIM // 16):
                    ys = yb * 16
                    s0 = pv[x, 0, pl.ds(ys, 16)] * _PROJ_W
                    s1 = pv[x, 1, pl.ds(ys, 16)] * _PROJ_W + NUM_HEADS
                    dvec = dv[x, pl.ds(ys, 16)]
                    rvec = jnp.where(dvec >= 2, jnp.float32(0.5),
                                     jnp.float32(1.0))
                    for h in range(NUM_HEADS):
                        av = plsc.load_gather(projv, [s0 + h])
                        bv = plsc.load_gather(projv, [s1 + h])
                        outv[x * NUM_HEADS + h, pl.ds(ys, 16)] = \
                            (av + bv) * rvec
            pltpu.sync_copy(
                outv,
                out_hbm.at[pl.ds(
                    pl.multiple_of((i * MAX_NODES + x0) * NUM_HEADS, 8),
                    _CHUNK_ROWS)])

        @pl.when(wid >= _NW - 2)
        def _fill():
            half = wid - (_NW - 2)
            neg = jnp.full((16,), -1000.0, dtype=jnp.float32)

            def body(r, _):
                for c in range(FEAT_DIM // 16):
                    outv[r, pl.ds(c * 16, 16)] = neg
                return _

            lax.fori_loop(0, _CHUNK_ROWS, body, None)
            pltpu.sync_copy(
                outv,
                out_hbm.at[pl.ds(
                    pl.multiple_of(((N_GRAPH - 1) * MAX_NODES
                                    + half * _XH) * NUM_HEADS, 8),
                    _CHUNK_ROWS)])

    return k(proj_flat, path_t, dist)


def kernel(edge_feat, dist, path, emb_weight):
    proj = _proj_matmul(edge_feat, emb_weight)
    proj_flat = proj.reshape(-1)
    path_t = jnp.transpose(path, (0, 1, 3, 2))
    out2d = _sc_combine(proj_flat, path_t, dist)
    out = out2d.reshape(N_GRAPH, MAX_NODES, NUM_HEADS, MAX_NODES)
    return jnp.transpose(out, (0, 1, 3, 2))
